# baseline (speedup 1.0000x reference)
import jax
import jax.numpy as jnp
from jax.experimental import pallas as pl
from jax.experimental.pallas import tpu as pltpu

_TB = 40


def _fused_kernel(x_ref, dep_ref, w1a_ref, w1d_ref, b1_ref, wa_ref,
                  wmsg_ref, b2_ref, wzrh_ref, uzr_ref, uh_ref, bzrh_ref,
                  wc_ref, bc_ref, out_ref, seq_s, x_s, outs_s):
    i = pl.program_id(0)
    nb = pl.num_programs(0) - 1
    TB, N, D = x_ref.shape
    d = wa_ref.shape[0]
    T = seq_s.shape[0]

    @pl.when(i < nb)
    def _spatial():
        x = x_ref[...].reshape(TB * N, D)
        h = jnp.dot(x, w1a_ref[...], preferred_element_type=jnp.float32)
        h = h + dep_ref[...] * w1d_ref[...] + b1_ref[...]
        h = jnp.maximum(h, 0.0)
        h3 = h.reshape(TB, N, d)
        s = jnp.sum(h3, axis=1) * (1.0 / (N - 1))
        svec = jnp.dot(s, wmsg_ref[...], preferred_element_type=jnp.float32)
        h2 = jnp.dot(h, wa_ref[...], preferred_element_type=jnp.float32)
        h2 = h2.reshape(TB, N, d) + svec[:, None, :] + b2_ref[...][None, :, :]
        h2 = jnp.maximum(h2, 0.0)
        seq_s[pl.ds(i * TB, TB), :] = jnp.mean(h2, axis=1)

    @pl.when(i == nb)
    def _gru():
        x_s[...] = jnp.dot(seq_s[...], wzrh_ref[...],
                           preferred_element_type=jnp.float32) + bzrh_ref[...]

        def step(t, h):
            xt = x_s[pl.ds(t, 1), :]
            zr = jnp.dot(h, uzr_ref[...], preferred_element_type=jnp.float32)
            z = jax.nn.sigmoid(xt[:, 0:d] + zr[:, 0:d])
            r = jax.nn.sigmoid(xt[:, d:2 * d] + zr[:, d:2 * d])
            hh_pre = xt[:, 2 * d:3 * d] + jnp.dot(
                r * h, uh_ref[...], preferred_element_type=jnp.float32)
            hh = jnp.tanh(hh_pre)
            hnew = h + z * (hh - h)
            outs_s[pl.ds(t, 1), :] = hnew
            return hnew

        h0 = jnp.zeros((1, d), dtype=jnp.float32)
        jax.lax.fori_loop(0, T, step, h0)
        logits = jnp.dot(outs_s[...], wc_ref[...],
                         preferred_element_type=jnp.float32) + bc_ref[...]
        out_ref[...] = jax.nn.sigmoid(logits)


def kernel(object_features, object_depths, W1, b1, W_self, W_msg, b2,
           W_z, U_z, b_z, W_r, U_r, b_r, W_h, U_h, b_h, Wc, bc):
    T, N, D = object_features.shape
    d = W_self.shape[0]
    nb = T // _TB

    W1a = W1[:D]
    w1d = W1[D:D + 1]
    b1r = b1.reshape(1, d)
    Wa = W_self - W_msg * (1.0 / (N - 1))
    b2r = b2.reshape(1, d)
    dep = object_depths.reshape(T * N, 1)
    Wzrh = jnp.concatenate([W_z, W_r, W_h], axis=1)
    bzrh = jnp.concatenate([b_z, b_r, b_h]).reshape(1, 3 * d)
    Uzr = jnp.concatenate([U_z, U_r], axis=1)
    bcr = bc.reshape(1, 1)

    clamp = lambda i: jnp.minimum(i, nb - 1)
    probs2d = pl.pallas_call(
        _fused_kernel,
        grid=(nb + 1,),
        in_specs=[
            pl.BlockSpec((_TB, N, D), lambda i: (clamp(i), 0, 0)),
            pl.BlockSpec((_TB * N, 1), lambda i: (clamp(i), 0)),
            pl.BlockSpec((D, d), lambda i: (0, 0)),
            pl.BlockSpec((1, d), lambda i: (0, 0)),
            pl.BlockSpec((1, d), lambda i: (0, 0)),
            pl.BlockSpec((d, d), lambda i: (0, 0)),
            pl.BlockSpec((d, d), lambda i: (0, 0)),
            pl.BlockSpec((1, d), lambda i: (0, 0)),
            pl.BlockSpec((d, 3 * d), lambda i: (0, 0)),
            pl.BlockSpec((d, 2 * d), lambda i: (0, 0)),
            pl.BlockSpec((d, d), lambda i: (0, 0)),
            pl.BlockSpec((1, 3 * d), lambda i: (0, 0)),
            pl.BlockSpec((d, 1), lambda i: (0, 0)),
            pl.BlockSpec((1, 1), lambda i: (0, 0)),
        ],
        out_specs=pl.BlockSpec((T, 1), lambda i: (0, 0)),
        out_shape=jax.ShapeDtypeStruct((T, 1), jnp.float32),
        scratch_shapes=[
            pltpu.VMEM((T, d), jnp.float32),
            pltpu.VMEM((T, 3 * d), jnp.float32),
            pltpu.VMEM((T, d), jnp.float32),
        ],
        compiler_params=pltpu.CompilerParams(
            dimension_semantics=("arbitrary",),
        ),
    )(object_features, dep, W1a, w1d, b1r, Wa, W_msg, b2r,
      Wzrh, Uzr, U_h, bzrh, Wc, bcr)

    probs = probs2d.reshape(T)
    uncertainty = jnp.zeros_like(probs)
    return (probs, uncertainty)


# speedup vs baseline: 1.1446x; 1.1446x over previous
import jax
import jax.numpy as jnp
from jax.experimental import pallas as pl
from jax.experimental.pallas import tpu as pltpu

_TB = 8


def _fused_kernel(x_ref, dep_ref, w1a_ref, w1d_ref, b1_ref, wa_ref,
                  wmsg_ref, b2_ref, wzrh_ref, uzr_ref, uh_ref, bzrh_ref,
                  wc_ref, bc_ref, out_ref, x_s, outs_s, h_s):
    i = pl.program_id(0)
    nb = pl.num_programs(0) - 1
    TB, N, D = x_ref.shape
    d = wa_ref.shape[0]

    base = jnp.maximum(i - 1, 0) * TB
    h = jnp.where(i <= 1, 0.0, h_s[...])
    for t in range(TB):
        xt = x_s[pl.ds(base + t, 1), :]
        zr = jnp.dot(h, uzr_ref[...], preferred_element_type=jnp.float32)
        z = jax.nn.sigmoid(xt[:, 0:d] + zr[:, 0:d])
        r = jax.nn.sigmoid(xt[:, d:2 * d] + zr[:, d:2 * d])
        hh = jnp.tanh(xt[:, 2 * d:3 * d] + jnp.dot(
            r * h, uh_ref[...], preferred_element_type=jnp.float32))
        h = h + z * (hh - h)
        outs_s[pl.ds(base + t, 1), :] = h
    h_s[...] = h

    x = x_ref[...].reshape(TB * N, D)
    hs = jnp.dot(x, w1a_ref[...], preferred_element_type=jnp.float32)
    hs = jnp.maximum(hs + dep_ref[...] * w1d_ref[...] + b1_ref[...], 0.0)
    h3 = hs.reshape(TB, N, d)
    s = jnp.sum(h3, axis=1) * (1.0 / (N - 1))
    svec = jnp.dot(s, wmsg_ref[...], preferred_element_type=jnp.float32)
    h2 = jnp.dot(hs, wa_ref[...], preferred_element_type=jnp.float32)
    h2 = h2.reshape(TB, N, d) + svec[:, None, :] + b2_ref[...][None, :, :]
    pooled = jnp.mean(jnp.maximum(h2, 0.0), axis=1)
    xb = jnp.dot(pooled, wzrh_ref[...],
                 preferred_element_type=jnp.float32) + bzrh_ref[...]
    x_s[pl.ds(jnp.minimum(i, nb - 1) * TB, TB), :] = xb

    @pl.when(i == nb)
    def _classifier():
        logits = jnp.dot(outs_s[...], wc_ref[...],
                         preferred_element_type=jnp.float32) + bc_ref[...]
        out_ref[...] = jax.nn.sigmoid(logits)


def kernel(object_features, object_depths, W1, b1, W_self, W_msg, b2,
           W_z, U_z, b_z, W_r, U_r, b_r, W_h, U_h, b_h, Wc, bc):
    T, N, D = object_features.shape
    d = W_self.shape[0]
    nb = T // _TB

    W1a = W1[:D]
    w1d = W1[D:D + 1]
    b1r = b1.reshape(1, d)
    Wa = W_self - W_msg * (1.0 / (N - 1))
    b2r = b2.reshape(1, d)
    dep = object_depths.reshape(T * N, 1)
    Wzrh = jnp.concatenate([W_z, W_r, W_h], axis=1)
    bzrh = jnp.concatenate([b_z, b_r, b_h]).reshape(1, 3 * d)
    Uzr = jnp.concatenate([U_z, U_r], axis=1)
    bcr = bc.reshape(1, 1)

    clamp = lambda i: jnp.minimum(i, nb - 1)
    probs2d = pl.pallas_call(
        _fused_kernel,
        grid=(nb + 1,),
        in_specs=[
            pl.BlockSpec((_TB, N, D), lambda i: (clamp(i), 0, 0)),
            pl.BlockSpec((_TB * N, 1), lambda i: (clamp(i), 0)),
            pl.BlockSpec((D, d), lambda i: (0, 0)),
            pl.BlockSpec((1, d), lambda i: (0, 0)),
            pl.BlockSpec((1, d), lambda i: (0, 0)),
            pl.BlockSpec((d, d), lambda i: (0, 0)),
            pl.BlockSpec((d, d), lambda i: (0, 0)),
            pl.BlockSpec((1, d), lambda i: (0, 0)),
            pl.BlockSpec((d, 3 * d), lambda i: (0, 0)),
            pl.BlockSpec((d, 2 * d), lambda i: (0, 0)),
            pl.BlockSpec((d, d), lambda i: (0, 0)),
            pl.BlockSpec((1, 3 * d), lambda i: (0, 0)),
            pl.BlockSpec((d, 1), lambda i: (0, 0)),
            pl.BlockSpec((1, 1), lambda i: (0, 0)),
        ],
        out_specs=pl.BlockSpec((T, 1), lambda i: (0, 0)),
        out_shape=jax.ShapeDtypeStruct((T, 1), jnp.float32),
        scratch_shapes=[
            pltpu.VMEM((T, 3 * d), jnp.float32),
            pltpu.VMEM((T, d), jnp.float32),
            pltpu.VMEM((1, d), jnp.float32),
        ],
        compiler_params=pltpu.CompilerParams(
            dimension_semantics=("arbitrary",),
        ),
    )(object_features, dep, W1a, w1d, b1r, Wa, W_msg, b2r,
      Wzrh, Uzr, U_h, bzrh, Wc, bcr)

    probs = probs2d.reshape(T)
    uncertainty = jnp.zeros_like(probs)
    return (probs, uncertainty)
